# baseline (speedup 1.0000x reference)
import functools

import jax
import jax.numpy as jnp
from jax.experimental import pallas as pl
from jax.experimental.pallas import tpu as pltpu

_NUM_EXPERTS = 64
_TOP_K = 2
_HIDDEN = 1024
_INTER = 512
_TOKENS = 32
_P = _TOKENS * _TOP_K


def _routing_kernel(logits_ref, eid_ref, isf_ref, wmat_ref):
    l = logits_ref[...]
    T, E = l.shape
    col = jax.lax.broadcasted_iota(jnp.int32, (T, E), 1).astype(jnp.float32)

    m1 = jnp.max(l, axis=1, keepdims=True)
    a1 = jnp.min(jnp.where(l >= m1, col, jnp.float32(E)), axis=1, keepdims=True)
    lm = jnp.where(col == a1, -jnp.inf, l)
    m2 = jnp.max(lm, axis=1, keepdims=True)
    a2 = jnp.min(jnp.where(lm >= m2, col, jnp.float32(E)), axis=1, keepdims=True)

    w1 = 1.0 / (1.0 + jnp.exp(m2 - m1))
    w2 = 1.0 - w1

    wmat_ref[...] = jnp.where(col == a1, w1, 0.0) + jnp.where(col == a2, w2, 0.0)

    eid_col = jnp.concatenate([a1, a2], axis=0)
    P = 2 * T
    pair_iota = jax.lax.broadcasted_iota(jnp.int32, (P, 1), 0).astype(jnp.float32)
    c_col = eid_col * P + pair_iota

    A = jnp.broadcast_to(c_col, (P, P))
    B = jnp.transpose(A)
    rank_col = jnp.sum((B < A).astype(jnp.float32), axis=1, keepdims=True)

    R = jnp.transpose(jnp.broadcast_to(rank_col, (P, P)))
    p_iota = jax.lax.broadcasted_iota(jnp.int32, (P, P), 0).astype(jnp.float32)
    S = (R == p_iota).astype(jnp.float32)
    S1 = (R == (p_iota - 1.0)).astype(jnp.float32)

    dot = functools.partial(
        jax.lax.dot, precision=jax.lax.Precision.HIGHEST,
        preferred_element_type=jnp.float32)
    sorted_eid = dot(S, eid_col)
    prev_eid = dot(S1, eid_col)
    first = jnp.logical_or(sorted_eid != prev_eid, p_iota[:, :1] == 0.0)
    eid_ref[...] = sorted_eid.astype(jnp.int32)
    isf_ref[...] = first.astype(jnp.int32)


def _moe_kernel(eid_s, isf_s, wmat_ref, x_ref, w13_ref, w2_ref, out_ref):
    p = pl.program_id(0)

    @pl.when(p == 0)
    def _():
        out_ref[...] = jnp.zeros_like(out_ref)

    @pl.when(isf_s[p] == 1)
    def _():
        w13e = w13_ref[0]
        gu = jax.lax.dot_general(
            x_ref[...], w13e, (((1,), (1,)), ((), ())),
            preferred_element_type=jnp.float32)
        gate = gu[:, :_INTER]
        up = gu[:, _INTER:]
        inter = gate * jax.lax.logistic(gate) * up
        down = jax.lax.dot_general(
            inter, w2_ref[0], (((1,), (1,)), ((), ())),
            preferred_element_type=jnp.float32)
        e = eid_s[p]
        iota_e = jax.lax.broadcasted_iota(jnp.int32, (_NUM_EXPERTS, 1), 0)
        onehot = (iota_e == e).astype(jnp.float32)
        wcol = jax.lax.dot(
            wmat_ref[...], onehot,
            preferred_element_type=jnp.float32)
        out_ref[...] = out_ref[...] + wcol * down


def kernel(x, router_logits, w13, w2):
    eid_c, isf_c, wmat = pl.pallas_call(
        _routing_kernel,
        out_shape=[
            jax.ShapeDtypeStruct((_P, 1), jnp.int32),
            jax.ShapeDtypeStruct((_P, 1), jnp.int32),
            jax.ShapeDtypeStruct((_TOKENS, _NUM_EXPERTS), jnp.float32),
        ],
    )(router_logits.astype(jnp.float32))
    eid = jnp.arange(64, dtype=jnp.int32)
    isf = jnp.ones((64,), dtype=jnp.int32)

    grid_spec = pltpu.PrefetchScalarGridSpec(
        num_scalar_prefetch=2,
        grid=(_P,),
        in_specs=[
            pl.BlockSpec((_TOKENS, _NUM_EXPERTS), lambda p, e, f: (0, 0)),
            pl.BlockSpec((_TOKENS, _HIDDEN), lambda p, e, f: (0, 0)),
            pl.BlockSpec((1, 2 * _INTER, _HIDDEN), lambda p, e, f: (e[p], 0, 0)),
            pl.BlockSpec((1, _HIDDEN, _INTER), lambda p, e, f: (e[p], 0, 0)),
        ],
        out_specs=pl.BlockSpec((_TOKENS, _HIDDEN), lambda p, e, f: (0, 0)),
    )
    out = pl.pallas_call(
        _moe_kernel,
        grid_spec=grid_spec,
        out_shape=jax.ShapeDtypeStruct((_TOKENS, _HIDDEN), jnp.float32),
        compiler_params=pltpu.CompilerParams(
            dimension_semantics=("arbitrary",)),
    )(eid, isf, wmat, x, w13, w2)
    return out.astype(x.dtype)


# speedup vs baseline: 1.2990x; 1.2990x over previous
import functools

import jax
import jax.numpy as jnp
from jax.experimental import pallas as pl
from jax.experimental.pallas import tpu as pltpu

_NUM_EXPERTS = 64
_TOP_K = 2
_HIDDEN = 1024
_INTER = 512
_TOKENS = 32
_P = _TOKENS * _TOP_K


def _routing_kernel(logits_ref, eid_ref, isf_ref, wmat_ref):
    l = logits_ref[...]
    T, E = l.shape
    col = jax.lax.broadcasted_iota(jnp.int32, (T, E), 1).astype(jnp.float32)

    m1 = jnp.max(l, axis=1, keepdims=True)
    a1 = jnp.min(jnp.where(l >= m1, col, jnp.float32(E)), axis=1, keepdims=True)
    lm = jnp.where(col == a1, -jnp.inf, l)
    m2 = jnp.max(lm, axis=1, keepdims=True)
    a2 = jnp.min(jnp.where(lm >= m2, col, jnp.float32(E)), axis=1, keepdims=True)

    w1 = 1.0 / (1.0 + jnp.exp(m2 - m1))
    w2 = 1.0 - w1

    wmat_ref[...] = jnp.where(col == a1, w1, 0.0) + jnp.where(col == a2, w2, 0.0)

    eid_col = jnp.concatenate([a1, a2], axis=0)
    P = 2 * T
    pair_iota = jax.lax.broadcasted_iota(jnp.int32, (P, 1), 0).astype(jnp.float32)
    c_col = eid_col * P + pair_iota

    A = jnp.broadcast_to(c_col, (P, P))
    B = jnp.transpose(A)
    rank_col = jnp.sum((B < A).astype(jnp.float32), axis=1, keepdims=True)

    R = jnp.transpose(jnp.broadcast_to(rank_col, (P, P)))
    p_iota = jax.lax.broadcasted_iota(jnp.int32, (P, P), 0).astype(jnp.float32)
    S = (R == p_iota).astype(jnp.float32)
    S1 = (R == (p_iota - 1.0)).astype(jnp.float32)

    dot = functools.partial(
        jax.lax.dot, precision=jax.lax.Precision.HIGHEST,
        preferred_element_type=jnp.float32)
    sorted_eid = dot(S, eid_col)
    prev_eid = dot(S1, eid_col)
    p_col = p_iota[:, :1]
    first = jnp.logical_or(sorted_eid != prev_eid, p_col == 0.0)
    first_f = first.astype(jnp.float32)

    q_iota = jax.lax.broadcasted_iota(jnp.int32, (P, P), 1).astype(jnp.float32)
    ltri = (q_iota <= p_iota).astype(jnp.float32)
    urank = dot(ltri, first_f) - 1.0
    ucount = jnp.sum(first_f, axis=0, keepdims=True)
    clamp_col = jnp.minimum(p_col, jnp.broadcast_to(ucount, (P, 1)) - 1.0)
    r_row = jnp.transpose(jnp.broadcast_to(urank, (P, P)))
    f_row = jnp.transpose(jnp.broadcast_to(first_f, (P, P)))
    M2 = (r_row == clamp_col).astype(jnp.float32) * f_row
    uniq_eid = dot(M2, sorted_eid)
    eid_ref[...] = uniq_eid.astype(jnp.int32)
    isf_ref[...] = (p_col < jnp.broadcast_to(ucount, (P, 1))).astype(jnp.int32)


def _moe_kernel(eid_s, isf_s, wmat_ref, x_ref, w13_ref, w2_ref, out_ref):
    p = pl.program_id(0)

    @pl.when(p == 0)
    def _():
        out_ref[...] = jnp.zeros_like(out_ref)

    @pl.when(isf_s[p] == 1)
    def _():
        w13e = w13_ref[0]
        gu = jax.lax.dot_general(
            x_ref[...], w13e, (((1,), (1,)), ((), ())),
            preferred_element_type=jnp.float32)
        gate = gu[:, :_INTER]
        up = gu[:, _INTER:]
        inter = gate * jax.lax.logistic(gate) * up
        down = jax.lax.dot_general(
            inter, w2_ref[0], (((1,), (1,)), ((), ())),
            preferred_element_type=jnp.float32)
        e = eid_s[p]
        iota_e = jax.lax.broadcasted_iota(jnp.int32, (_NUM_EXPERTS, 1), 0)
        onehot = (iota_e == e).astype(jnp.float32)
        wcol = jax.lax.dot(
            wmat_ref[...], onehot,
            preferred_element_type=jnp.float32)
        out_ref[...] = out_ref[...] + wcol * down


def kernel(x, router_logits, w13, w2):
    eid_c, isf_c, wmat = pl.pallas_call(
        _routing_kernel,
        out_shape=[
            jax.ShapeDtypeStruct((_P, 1), jnp.int32),
            jax.ShapeDtypeStruct((_P, 1), jnp.int32),
            jax.ShapeDtypeStruct((_TOKENS, _NUM_EXPERTS), jnp.float32),
        ],
    )(router_logits.astype(jnp.float32))
    eid = eid_c.reshape(-1)
    isf = isf_c.reshape(-1)

    grid_spec = pltpu.PrefetchScalarGridSpec(
        num_scalar_prefetch=2,
        grid=(_P,),
        in_specs=[
            pl.BlockSpec((_TOKENS, _NUM_EXPERTS), lambda p, e, f: (0, 0)),
            pl.BlockSpec((_TOKENS, _HIDDEN), lambda p, e, f: (0, 0)),
            pl.BlockSpec((1, 2 * _INTER, _HIDDEN), lambda p, e, f: (e[p], 0, 0)),
            pl.BlockSpec((1, _HIDDEN, _INTER), lambda p, e, f: (e[p], 0, 0)),
        ],
        out_specs=pl.BlockSpec((_TOKENS, _HIDDEN), lambda p, e, f: (0, 0)),
    )
    out = pl.pallas_call(
        _moe_kernel,
        grid_spec=grid_spec,
        out_shape=jax.ShapeDtypeStruct((_TOKENS, _HIDDEN), jnp.float32),
        compiler_params=pltpu.CompilerParams(
            dimension_semantics=("arbitrary",)),
    )(eid, isf, wmat, x, w13, w2)
    return out.astype(x.dtype)
